# baseline (speedup 1.0000x reference)
import jax
import jax.numpy as jnp
from jax.experimental import pallas as pl

GRID_N = 27
PATCHES = GRID_N * GRID_N
BATCH_BLOCK = 4


def _body(x_ref, row_ref, col_ref, o_ref):
    row = row_ref[...]
    col = col_ref[...]
    rr = jnp.reshape(
        jax.lax.broadcast_in_dim(row, (GRID_N, GRID_N, row.shape[-1]), (0, 2)),
        (PATCHES, row.shape[-1]),
    )
    cc = jnp.reshape(
        jax.lax.broadcast_in_dim(col, (GRID_N, GRID_N, col.shape[-1]), (1, 2)),
        (PATCHES, col.shape[-1]),
    )
    pos = rr + cc
    o_ref[...] = x_ref[...] + pos[None, :, :]


def kernel(x, row_emb, col_emb):
    b, p, d = x.shape
    grid = (b // BATCH_BLOCK,)
    return pl.pallas_call(
        _body,
        grid=grid,
        in_specs=[
            pl.BlockSpec((BATCH_BLOCK, p, d), lambda i: (i, 0, 0)),
            pl.BlockSpec((GRID_N, d), lambda i: (0, 0)),
            pl.BlockSpec((GRID_N, d), lambda i: (0, 0)),
        ],
        out_specs=pl.BlockSpec((BATCH_BLOCK, p, d), lambda i: (i, 0, 0)),
        out_shape=jax.ShapeDtypeStruct(x.shape, x.dtype),
    )(x, row_emb, col_emb)


# speedup vs baseline: 1.0006x; 1.0006x over previous
import jax
import jax.numpy as jnp
from jax.experimental import pallas as pl
from jax.experimental.pallas import tpu as pltpu

GRID_N = 27
PATCHES = GRID_N * GRID_N
BATCH_BLOCK = 4


def _body(x_ref, row_ref, col_ref, o_ref, pos_ref):
    i = pl.program_id(0)

    @pl.when(i == 0)
    def _():
        row = row_ref[...]
        col = col_ref[...]
        d = row.shape[-1]
        rr = jnp.reshape(
            jax.lax.broadcast_in_dim(row, (GRID_N, GRID_N, d), (0, 2)),
            (PATCHES, d),
        )
        cc = jnp.reshape(
            jax.lax.broadcast_in_dim(col, (GRID_N, GRID_N, d), (1, 2)),
            (PATCHES, d),
        )
        pos_ref[...] = rr + cc

    o_ref[...] = x_ref[...] + pos_ref[...][None, :, :]


def kernel(x, row_emb, col_emb):
    b, p, d = x.shape
    grid = (b // BATCH_BLOCK,)
    return pl.pallas_call(
        _body,
        grid=grid,
        in_specs=[
            pl.BlockSpec((BATCH_BLOCK, p, d), lambda i: (i, 0, 0)),
            pl.BlockSpec((GRID_N, d), lambda i: (0, 0)),
            pl.BlockSpec((GRID_N, d), lambda i: (0, 0)),
        ],
        out_specs=pl.BlockSpec((BATCH_BLOCK, p, d), lambda i: (i, 0, 0)),
        out_shape=jax.ShapeDtypeStruct(x.shape, x.dtype),
        scratch_shapes=[pltpu.VMEM((PATCHES, d), x.dtype)],
        compiler_params=pltpu.CompilerParams(
            dimension_semantics=("arbitrary",),
        ),
    )(x, row_emb, col_emb)
